# baseline (speedup 1.0000x reference)
import functools

import jax
import jax.numpy as jnp
from jax import lax
from jax.experimental import pallas as pl
from jax.experimental.pallas import tpu as pltpu
from jax.experimental.pallas import tpu_sc as plsc

NC = 2
NS = 16
LANES = 128


def _make_degree_kernel(num_nodes, hist_n, chunks_per_worker):
    mesh = plsc.VectorSubcoreMesh(
        core_axis_name="c", subcore_axis_name="s",
        num_cores=NC, num_subcores=NS)

    zslice = hist_n // NS

    @functools.partial(
        pl.kernel,
        mesh=mesh,
        out_type=jax.ShapeDtypeStruct((NC, hist_n), jnp.float32),
        scratch_types=[
            pltpu.VMEM((chunks_per_worker, LANES), jnp.int32),
            pltpu.VMEM((LANES,), jnp.float32),
            pltpu.VMEM((zslice,), jnp.float32),
            pltpu.VMEM_SHARED((hist_n,), jnp.float32),
        ],
    )
    def degree_kernel(tgt_hbm, out_hbm, tgt_v, ones_v, zer_v, hist_sh):
        c = lax.axis_index("c")
        s = lax.axis_index("s")
        wid = c * NS + s

        def fill(i, _):
            ones_v[pl.ds(i * 16, 16)] = jnp.ones((16,), jnp.float32)
            return 0
        lax.fori_loop(0, LANES // 16, fill, 0)

        def zfill(i, _):
            zer_v[pl.ds(i * 16, 16)] = jnp.zeros((16,), jnp.float32)
            return 0
        lax.fori_loop(0, zslice // 16, zfill, 0)
        pltpu.sync_copy(zer_v, hist_sh.at[pl.ds(s * zslice, zslice)])

        pltpu.sync_copy(tgt_hbm.at[wid], tgt_v)
        plsc.subcore_barrier()

        def body(j, _):
            pltpu.sync_copy(ones_v, hist_sh.at[tgt_v.at[j]], add=True)
            return 0
        lax.fori_loop(0, chunks_per_worker, body, 0)

        plsc.subcore_barrier()

        @pl.when(s == 0)
        def _():
            pltpu.sync_copy(hist_sh, out_hbm.at[c])

    return degree_kernel


def _make_aggregate_kernel(num_nodes, dout, chunks_per_tile):
    half = num_nodes // NC
    acc_rows = -(-(half + NS * 16) // (NS * 8)) * (NS * 8)
    zrows = acc_rows // NS

    mesh = plsc.VectorSubcoreMesh(
        core_axis_name="c", subcore_axis_name="s",
        num_cores=NC, num_subcores=NS)

    G = 8
    ngroups = chunks_per_tile // G
    ge = G * LANES
    CH = 96
    R = 3
    cap = ge + CH
    crows = -(-cap // CH)

    last_rows = half - (NS - 1) * zrows
    assert 0 < last_rows <= zrows and last_rows % 8 == 0

    @functools.partial(
        pl.kernel,
        mesh=mesh,
        out_type=jax.ShapeDtypeStruct((num_nodes, dout), jnp.float32),
        compiler_params=pltpu.CompilerParams(needs_layout_passes=False),
        scratch_types=[
            pltpu.VMEM((2, G, LANES), jnp.int32),
            pltpu.VMEM((2, G, LANES), jnp.int32),
            pltpu.VMEM((cap,), jnp.int32),
            pltpu.VMEM((cap,), jnp.int32),
            pltpu.VMEM((crows, CH), jnp.int32),
            [pltpu.VMEM((CH, dout), jnp.float32)] * R,
            [pltpu.SemaphoreType.DMA] * R,
            [pltpu.SemaphoreType.DMA] * R,
            pltpu.SemaphoreType.DMA,
            pltpu.VMEM_SHARED((acc_rows, dout), jnp.float32),
        ],
    )
    def agg_kernel(hn_hbm, src_hbm, tgt_hbm, out_hbm,
                   src_v, tgt_v, csrc, cstmp, cscat, bufs,
                   sems_g, sems_s, sem_i, acc_sh):
        c = lax.axis_index("c")
        s = lax.axis_index("s")
        base = c * half
        dummy_base = half + s * 16

        def zfill(i, _):
            def zrow(k, _):
                bufs[0][i, pl.ds(k * 16, 16)] = jnp.zeros((16,), jnp.float32)
                return 0
            lax.fori_loop(0, dout // 16, zrow, 0)
            return 0
        lax.fori_loop(0, CH, zfill, 0)
        for r in range(0, zrows, CH):
            sz = min(CH, zrows - r)
            pltpu.sync_copy(bufs[0].at[pl.ds(0, sz)],
                            acc_sh.at[pl.ds(s * zrows + r, sz)])

        def idx_start(g, slot):
            off = pl.multiple_of(g * G, 8)
            pltpu.make_async_copy(
                src_hbm.at[s, pl.ds(off, G)], src_v.at[slot], sem_i).start()
            pltpu.make_async_copy(
                tgt_hbm.at[s, pl.ds(off, G)], tgt_v.at[slot], sem_i).start()

        def idx_wait():
            pltpu.make_async_copy(
                src_hbm.at[s, pl.ds(0, G)], src_v.at[0], sem_i).wait()
            pltpu.make_async_copy(
                tgt_hbm.at[s, pl.ds(0, G)], tgt_v.at[0], sem_i).wait()

        def gather(j, q):
            pltpu.make_async_copy(
                hn_hbm.at[csrc.at[pl.ds(j * CH, CH)]], bufs[q], sems_g[q]
            ).start()

        def gwait(q):
            pltpu.make_async_copy(
                hn_hbm.at[csrc.at[pl.ds(0, CH)]], bufs[q], sems_g[q]).wait()

        def scat_start(j, q):
            pltpu.async_copy(bufs[q], acc_sh.at[cscat.at[j]], sems_s[q],
                             add=True)

        def swait(q):
            pltpu.make_async_copy(
                bufs[q], acc_sh.at[cscat.at[0]], sems_s[q]).wait()

        idx_start(0, 0)
        idx_wait()
        plsc.subcore_barrier()

        true16 = jnp.ones((16,), jnp.bool_)
        zero16 = jnp.zeros((16,), jnp.int32)

        def group(g, _):
            slot = g & 1

            @pl.when(g + 1 < ngroups)
            def _():
                idx_start(g + 1, 1 - slot)

            def comp(i, p_v):
                for u in range(2):
                    v = i * 2 + u
                    j = v >> 3
                    off = (v & 7) * 16
                    t = tgt_v[slot, j, pl.ds(off, 16)]
                    sv = src_v[slot, j, pl.ds(off, 16)]
                    m = (t >= base) & (t < base + half)
                    pos = p_v + plsc.cumsum(m.astype(jnp.int32)) - 1
                    plsc.store_scatter(cstmp, [pos], t - base, mask=m)
                    plsc.store_scatter(csrc, [pos], sv, mask=m)
                    p_v = p_v + plsc.all_reduce_population_count(m)
                return p_v
            p_v = lax.fori_loop(0, ge // 32, comp, jnp.zeros((16,), jnp.int32))
            p = jnp.sum(p_v) >> 4

            padv = zero16 + dummy_base
            iota16 = jax.lax.iota(jnp.int32, 16)
            for q in range(CH // 16):
                padpos = p + q * 16 + iota16
                plsc.store_scatter(cstmp, [padpos], padv, mask=true16)
                plsc.store_scatter(csrc, [padpos], zero16, mask=true16)
            nch = (p + CH - 1) // CH

            def ccopy(jr, _):
                for k in range(CH // 16):
                    cscat[jr, pl.ds(k * 16, 16)] = (
                        cstmp[pl.ds(jr * CH + k * 16, 16)])
                return 0
            lax.fori_loop(0, nch, ccopy, 0)

            @pl.when(nch > 0)
            def _():
                gather(0, 0)

                def inner(i, _):
                    im = lax.rem(i, R)
                    for q in range(R):
                        nq = (q + 1) % R

                        @pl.when(im == q)
                        def _(q=q, nq=nq):
                            @pl.when(i + 1 < nch)
                            def _():
                                @pl.when(i >= R - 1)
                                def _():
                                    swait(nq)
                                gather(i + 1, nq)
                            gwait(q)
                            scat_start(i, q)
                    return 0
                lax.fori_loop(0, nch, inner, 0)

                for q in range(R):
                    @pl.when(q < nch)
                    def _(q=q):
                        swait(q)

            @pl.when(g + 1 < ngroups)
            def _():
                idx_wait()
            return 0
        lax.fori_loop(0, ngroups, group, 0)

        plsc.subcore_barrier()

        @pl.when(s < NS - 1)
        def _():
            pltpu.sync_copy(
                acc_sh.at[pl.ds(s * zrows, zrows)],
                out_hbm.at[pl.ds(c * half + s * zrows, zrows)])

        @pl.when(s == NS - 1)
        def _():
            off = (NS - 1) * zrows
            pltpu.sync_copy(
                acc_sh.at[pl.ds(off, last_rows)],
                out_hbm.at[pl.ds(c * half + off, last_rows)])

    return agg_kernel


def _linear_body(x_ref, wt_ref, b_ref, deg_ref, h_ref, hn_ref):
    x = x_ref[...]
    h = jnp.dot(x, wt_ref[...], preferred_element_type=jnp.float32) + b_ref[...]
    dis = lax.rsqrt(deg_ref[...])
    h_ref[...] = h
    hn_ref[...] = h * dis


def _tc_linear(x, wt, b2, deg_col, rb):
    n = x.shape[0]
    din = x.shape[1]
    dout = wt.shape[1]
    grid = n // rb
    return pl.pallas_call(
        _linear_body,
        grid=(grid,),
        in_specs=[
            pl.BlockSpec((rb, din), lambda i: (i, 0)),
            pl.BlockSpec((din, dout), lambda i: (0, 0)),
            pl.BlockSpec((1, dout), lambda i: (0, 0)),
            pl.BlockSpec((rb, 1), lambda i: (i, 0)),
        ],
        out_specs=[
            pl.BlockSpec((rb, dout), lambda i: (i, 0)),
            pl.BlockSpec((rb, dout), lambda i: (i, 0)),
        ],
        out_shape=[
            jax.ShapeDtypeStruct((n, dout), jnp.float32),
            jax.ShapeDtypeStruct((n, dout), jnp.float32),
        ],
    )(x, wt, b2, deg_col)


def _finish_body(h_ref, agg_ref, deg_ref, g_ref, be_ref, o_ref):
    h = h_ref[...]
    dis = lax.rsqrt(deg_ref[...])
    y = h + dis * (agg_ref[...] + h)
    mean = jnp.mean(y, axis=1, keepdims=True)
    yc = y - mean
    var = jnp.mean(yc * yc, axis=1, keepdims=True)
    o = yc * lax.rsqrt(var + 1e-5) * g_ref[...] + be_ref[...]
    o_ref[...] = jnp.maximum(o, 0.0)


def _tc_finish(h, agg, deg_col, g2, be2, rb):
    n, dout = h.shape
    grid = n // rb
    return pl.pallas_call(
        _finish_body,
        grid=(grid,),
        in_specs=[
            pl.BlockSpec((rb, dout), lambda i: (i, 0)),
            pl.BlockSpec((rb, dout), lambda i: (i, 0)),
            pl.BlockSpec((rb, 1), lambda i: (i, 0)),
            pl.BlockSpec((1, dout), lambda i: (0, 0)),
            pl.BlockSpec((1, dout), lambda i: (0, 0)),
        ],
        out_specs=pl.BlockSpec((rb, dout), lambda i: (i, 0)),
        out_shape=jax.ShapeDtypeStruct((n, dout), jnp.float32),
    )(h, agg, deg_col, g2, be2)


def kernel(node_features, edge_index, W, b, gamma, beta):
    bs, n, din = node_features.shape
    dout = W.shape[0]
    nn = bs * n
    e = edge_index.shape[2]
    be = bs * e

    ei = edge_index.astype(jnp.int32)
    offs = (jnp.arange(bs, dtype=jnp.int32) * n)[:, None]
    src = (ei[:, 0, :] + offs).reshape(-1)
    tgt = (ei[:, 1, :] + offs).reshape(-1)
    x = node_features.reshape(nn, din).astype(jnp.float32)

    cpw = -(-be // (NC * NS * LANES))
    cpw = -(-cpw // 8) * 8
    be_pad = NC * NS * cpw * LANES
    cpt = be_pad // (NS * LANES)
    pad = be_pad - be
    srcp = jnp.concatenate([src, jnp.zeros((pad,), jnp.int32)])
    tgtp = jnp.concatenate([tgt, jnp.full((pad,), nn, jnp.int32)])

    hist_n = ((nn + 1 + 255) // 256) * 256

    degree_kernel = _make_degree_kernel(nn, hist_n, cpw)
    hist = degree_kernel(tgtp.reshape(NC * NS, cpw, LANES))
    deg_col = (hist[0, :nn] + hist[1, :nn] + 1.0).reshape(nn, 1)

    rb = 1000
    h, hn = _tc_linear(x, W.T, b.reshape(1, dout), deg_col, rb)

    agg_kernel = _make_aggregate_kernel(nn, dout, cpt)
    agg = agg_kernel(hn,
                     srcp.reshape(NS, cpt, LANES),
                     tgtp.reshape(NS, cpt, LANES))

    out = _tc_finish(h, agg, deg_col,
                     gamma.reshape(1, dout), beta.reshape(1, dout), rb)
    return out.reshape(bs, n, dout)


# speedup vs baseline: 1.7633x; 1.7633x over previous
import functools

import jax
import jax.numpy as jnp
from jax import lax
from jax.experimental import pallas as pl
from jax.experimental.pallas import tpu as pltpu
from jax.experimental.pallas import tpu_sc as plsc

NC = 2
NS = 16
LANES = 128


def _make_degree_kernel(num_nodes, hist_n, chunks_per_worker):
    mesh = plsc.VectorSubcoreMesh(
        core_axis_name="c", subcore_axis_name="s",
        num_cores=NC, num_subcores=NS)

    zslice = hist_n // NS

    @functools.partial(
        pl.kernel,
        mesh=mesh,
        out_type=jax.ShapeDtypeStruct((NC, hist_n), jnp.float32),
        scratch_types=[
            pltpu.VMEM((chunks_per_worker, LANES), jnp.int32),
            pltpu.VMEM((LANES,), jnp.float32),
            pltpu.VMEM((zslice,), jnp.float32),
            pltpu.VMEM_SHARED((hist_n,), jnp.float32),
        ],
    )
    def degree_kernel(tgt_hbm, out_hbm, tgt_v, ones_v, zer_v, hist_sh):
        c = lax.axis_index("c")
        s = lax.axis_index("s")
        wid = c * NS + s

        def fill(i, _):
            ones_v[pl.ds(i * 16, 16)] = jnp.ones((16,), jnp.float32)
            return 0
        lax.fori_loop(0, LANES // 16, fill, 0)

        def zfill(i, _):
            zer_v[pl.ds(i * 16, 16)] = jnp.zeros((16,), jnp.float32)
            return 0
        lax.fori_loop(0, zslice // 16, zfill, 0)
        pltpu.sync_copy(zer_v, hist_sh.at[pl.ds(s * zslice, zslice)])

        pltpu.sync_copy(tgt_hbm.at[wid], tgt_v)
        plsc.subcore_barrier()

        def body(j, _):
            pltpu.sync_copy(ones_v, hist_sh.at[tgt_v.at[j]], add=True)
            return 0
        lax.fori_loop(0, chunks_per_worker, body, 0)

        plsc.subcore_barrier()

        @pl.when(s == 0)
        def _():
            pltpu.sync_copy(hist_sh, out_hbm.at[c])

    return degree_kernel


def _make_aggregate_kernel(num_nodes, dout, chunks_per_tile):
    half = num_nodes // NC
    acc_rows = -(-(half + NS * 16) // (NS * 8)) * (NS * 8)
    zrows = acc_rows // NS

    mesh = plsc.VectorSubcoreMesh(
        core_axis_name="c", subcore_axis_name="s",
        num_cores=NC, num_subcores=NS)

    G = 8
    ngroups = chunks_per_tile // G
    ge = G * LANES
    CH = LANES
    cap = ge + CH
    crows = -(-cap // CH)

    last_rows = half - (NS - 1) * zrows
    assert 0 < last_rows <= zrows and last_rows % 8 == 0

    @functools.partial(
        pl.kernel,
        mesh=mesh,
        out_type=jax.ShapeDtypeStruct((num_nodes, dout), jnp.float32),
        compiler_params=pltpu.CompilerParams(needs_layout_passes=False),
        scratch_types=[
            pltpu.VMEM((2, G, LANES), jnp.int32),
            pltpu.VMEM((2, G, LANES), jnp.int32),
            pltpu.VMEM((cap,), jnp.int32),
            pltpu.VMEM((cap,), jnp.int32),
            pltpu.VMEM((crows, CH), jnp.int32),
            [pltpu.VMEM((CH, dout), jnp.float32)] * 2,
            [pltpu.SemaphoreType.DMA] * 2,
            pltpu.SemaphoreType.DMA,
            pltpu.VMEM_SHARED((acc_rows, dout), jnp.float32),
        ],
    )
    def agg_kernel(hn_hbm, src_hbm, tgt_hbm, out_hbm,
                   src_v, tgt_v, csrc, cstmp, cscat, bufs,
                   sems_g, sem_i, acc_sh):
        c = lax.axis_index("c")
        s = lax.axis_index("s")
        base = c * half
        dummy_base = half + s * 16

        def zfill(i, _):
            def zrow(k, _):
                bufs[0][i, pl.ds(k * 16, 16)] = jnp.zeros((16,), jnp.float32)
                return 0
            lax.fori_loop(0, dout // 16, zrow, 0)
            return 0
        lax.fori_loop(0, CH, zfill, 0)
        for r in range(0, zrows, CH):
            sz = min(CH, zrows - r)
            pltpu.sync_copy(bufs[0].at[pl.ds(0, sz)],
                            acc_sh.at[pl.ds(s * zrows + r, sz)])

        def idx_start(g, slot):
            off = pl.multiple_of(g * G, 8)
            pltpu.make_async_copy(
                src_hbm.at[s, pl.ds(off, G)], src_v.at[slot], sem_i).start()
            pltpu.make_async_copy(
                tgt_hbm.at[s, pl.ds(off, G)], tgt_v.at[slot], sem_i).start()

        def idx_wait():
            pltpu.make_async_copy(
                src_hbm.at[s, pl.ds(0, G)], src_v.at[0], sem_i).wait()
            pltpu.make_async_copy(
                tgt_hbm.at[s, pl.ds(0, G)], tgt_v.at[0], sem_i).wait()

        def gather(j, q):
            pltpu.make_async_copy(
                hn_hbm.at[csrc.at[pl.ds(j * CH, CH)]], bufs[q], sems_g[q]
            ).start()

        def gwait(q):
            pltpu.make_async_copy(
                hn_hbm.at[csrc.at[pl.ds(0, CH)]], bufs[q], sems_g[q]).wait()

        def scat_add(j, q):
            pltpu.sync_copy(bufs[q], acc_sh.at[cscat.at[j]], add=True)

        idx_start(0, 0)
        idx_wait()
        plsc.subcore_barrier()

        true16 = jnp.ones((16,), jnp.bool_)
        zero16 = jnp.zeros((16,), jnp.int32)

        def group(g, _):
            slot = g & 1

            @pl.when(g + 1 < ngroups)
            def _():
                idx_start(g + 1, 1 - slot)

            def comp(i, p_v):
                for u in range(2):
                    v = i * 2 + u
                    j = v >> 3
                    off = (v & 7) * 16
                    t = tgt_v[slot, j, pl.ds(off, 16)]
                    sv = src_v[slot, j, pl.ds(off, 16)]
                    m = (t >= base) & (t < base + half)
                    pos = p_v + plsc.cumsum(m.astype(jnp.int32)) - 1
                    plsc.store_scatter(cstmp, [pos], t - base, mask=m)
                    plsc.store_scatter(csrc, [pos], sv, mask=m)
                    p_v = p_v + plsc.all_reduce_population_count(m)
                return p_v
            p_v = lax.fori_loop(0, ge // 32, comp, jnp.zeros((16,), jnp.int32))
            p = jnp.sum(p_v) >> 4

            padv = zero16 + dummy_base
            iota16 = jax.lax.iota(jnp.int32, 16)
            for q in range(CH // 16):
                padpos = p + q * 16 + iota16
                plsc.store_scatter(cstmp, [padpos], padv, mask=true16)
                plsc.store_scatter(csrc, [padpos], zero16, mask=true16)
            nch = (p + CH - 1) // CH

            def ccopy(jr, _):
                for k in range(CH // 16):
                    cscat[jr, pl.ds(k * 16, 16)] = (
                        cstmp[pl.ds(jr * CH + k * 16, 16)])
                return 0
            lax.fori_loop(0, nch, ccopy, 0)

            @pl.when(nch > 0)
            def _():
                gather(0, 0)

                def inner(i, _):
                    even = (i & 1) == 0

                    @pl.when(even)
                    def _():
                        @pl.when(i + 1 < nch)
                        def _():
                            gather(i + 1, 1)
                        gwait(0)
                        scat_add(i, 0)

                    @pl.when(jnp.logical_not(even))
                    def _():
                        @pl.when(i + 1 < nch)
                        def _():
                            gather(i + 1, 0)
                        gwait(1)
                        scat_add(i, 1)
                    return 0
                lax.fori_loop(0, nch, inner, 0)

            @pl.when(g + 1 < ngroups)
            def _():
                idx_wait()
            return 0
        lax.fori_loop(0, ngroups, group, 0)

        plsc.subcore_barrier()

        @pl.when(s < NS - 1)
        def _():
            pltpu.sync_copy(
                acc_sh.at[pl.ds(s * zrows, zrows)],
                out_hbm.at[pl.ds(c * half + s * zrows, zrows)])

        @pl.when(s == NS - 1)
        def _():
            off = (NS - 1) * zrows
            pltpu.sync_copy(
                acc_sh.at[pl.ds(off, last_rows)],
                out_hbm.at[pl.ds(c * half + off, last_rows)])

    return agg_kernel


def _linear_body(x_ref, wt_ref, b_ref, deg_ref, h_ref, hn_ref):
    x = x_ref[...]
    h = jnp.dot(x, wt_ref[...], preferred_element_type=jnp.float32) + b_ref[...]
    dis = lax.rsqrt(deg_ref[...])
    h_ref[...] = h
    hn_ref[...] = h * dis


def _tc_linear(x, wt, b2, deg_col, rb):
    n = x.shape[0]
    din = x.shape[1]
    dout = wt.shape[1]
    grid = n // rb
    return pl.pallas_call(
        _linear_body,
        grid=(grid,),
        in_specs=[
            pl.BlockSpec((rb, din), lambda i: (i, 0)),
            pl.BlockSpec((din, dout), lambda i: (0, 0)),
            pl.BlockSpec((1, dout), lambda i: (0, 0)),
            pl.BlockSpec((rb, 1), lambda i: (i, 0)),
        ],
        out_specs=[
            pl.BlockSpec((rb, dout), lambda i: (i, 0)),
            pl.BlockSpec((rb, dout), lambda i: (i, 0)),
        ],
        out_shape=[
            jax.ShapeDtypeStruct((n, dout), jnp.float32),
            jax.ShapeDtypeStruct((n, dout), jnp.float32),
        ],
    )(x, wt, b2, deg_col)


def _finish_body(h_ref, agg_ref, deg_ref, g_ref, be_ref, o_ref):
    h = h_ref[...]
    dis = lax.rsqrt(deg_ref[...])
    y = h + dis * (agg_ref[...] + h)
    mean = jnp.mean(y, axis=1, keepdims=True)
    yc = y - mean
    var = jnp.mean(yc * yc, axis=1, keepdims=True)
    o = yc * lax.rsqrt(var + 1e-5) * g_ref[...] + be_ref[...]
    o_ref[...] = jnp.maximum(o, 0.0)


def _tc_finish(h, agg, deg_col, g2, be2, rb):
    n, dout = h.shape
    grid = n // rb
    return pl.pallas_call(
        _finish_body,
        grid=(grid,),
        in_specs=[
            pl.BlockSpec((rb, dout), lambda i: (i, 0)),
            pl.BlockSpec((rb, dout), lambda i: (i, 0)),
            pl.BlockSpec((rb, 1), lambda i: (i, 0)),
            pl.BlockSpec((1, dout), lambda i: (0, 0)),
            pl.BlockSpec((1, dout), lambda i: (0, 0)),
        ],
        out_specs=pl.BlockSpec((rb, dout), lambda i: (i, 0)),
        out_shape=jax.ShapeDtypeStruct((n, dout), jnp.float32),
    )(h, agg, deg_col, g2, be2)


def kernel(node_features, edge_index, W, b, gamma, beta):
    bs, n, din = node_features.shape
    dout = W.shape[0]
    nn = bs * n
    e = edge_index.shape[2]
    be = bs * e

    ei = edge_index.astype(jnp.int32)
    offs = (jnp.arange(bs, dtype=jnp.int32) * n)[:, None]
    src = (ei[:, 0, :] + offs).reshape(-1)
    tgt = (ei[:, 1, :] + offs).reshape(-1)
    x = node_features.reshape(nn, din).astype(jnp.float32)

    cpw = -(-be // (NC * NS * LANES))
    cpw = -(-cpw // 8) * 8
    be_pad = NC * NS * cpw * LANES
    cpt = be_pad // (NS * LANES)
    pad = be_pad - be
    srcp = jnp.concatenate([src, jnp.zeros((pad,), jnp.int32)])
    tgtp = jnp.concatenate([tgt, jnp.full((pad,), nn, jnp.int32)])

    hist_n = ((nn + 1 + 255) // 256) * 256

    degree_kernel = _make_degree_kernel(nn, hist_n, cpw)
    hist = degree_kernel(tgtp.reshape(NC * NS, cpw, LANES))
    deg_col = (hist[0, :nn] + hist[1, :nn] + 1.0).reshape(nn, 1)

    rb = 1000
    h, hn = _tc_linear(x, W.T, b.reshape(1, dout), deg_col, rb)

    agg_kernel = _make_aggregate_kernel(nn, dout, cpt)
    agg = agg_kernel(hn,
                     srcp.reshape(NS, cpt, LANES),
                     tgtp.reshape(NS, cpt, LANES))

    out = _tc_finish(h, agg, deg_col,
                     gamma.reshape(1, dout), beta.reshape(1, dout), rb)
    return out.reshape(bs, n, dout)


# speedup vs baseline: 1.8271x; 1.0362x over previous
import functools

import jax
import jax.numpy as jnp
from jax import lax
from jax.experimental import pallas as pl
from jax.experimental.pallas import tpu as pltpu
from jax.experimental.pallas import tpu_sc as plsc

NC = 2
NS = 16
LANES = 128


def _make_degree_kernel(num_nodes, hist_n, chunks_per_worker):
    mesh = plsc.VectorSubcoreMesh(
        core_axis_name="c", subcore_axis_name="s",
        num_cores=NC, num_subcores=NS)

    zslice = hist_n // NS

    @functools.partial(
        pl.kernel,
        mesh=mesh,
        out_type=jax.ShapeDtypeStruct((NC, hist_n), jnp.float32),
        scratch_types=[
            pltpu.VMEM((chunks_per_worker, LANES), jnp.int32),
            pltpu.VMEM((LANES,), jnp.float32),
            pltpu.VMEM((zslice,), jnp.float32),
            pltpu.VMEM_SHARED((hist_n,), jnp.float32),
        ],
    )
    def degree_kernel(tgt_hbm, out_hbm, tgt_v, ones_v, zer_v, hist_sh):
        c = lax.axis_index("c")
        s = lax.axis_index("s")
        wid = c * NS + s

        def fill(i, _):
            ones_v[pl.ds(i * 16, 16)] = jnp.ones((16,), jnp.float32)
            return 0
        lax.fori_loop(0, LANES // 16, fill, 0)

        def zfill(i, _):
            zer_v[pl.ds(i * 16, 16)] = jnp.zeros((16,), jnp.float32)
            return 0
        lax.fori_loop(0, zslice // 16, zfill, 0)
        pltpu.sync_copy(zer_v, hist_sh.at[pl.ds(s * zslice, zslice)])

        pltpu.sync_copy(tgt_hbm.at[wid], tgt_v)
        plsc.subcore_barrier()

        def body(j, _):
            pltpu.sync_copy(ones_v, hist_sh.at[tgt_v.at[j]], add=True)
            return 0
        lax.fori_loop(0, chunks_per_worker, body, 0)

        plsc.subcore_barrier()

        @pl.when(s == 0)
        def _():
            pltpu.sync_copy(hist_sh, out_hbm.at[c])

    return degree_kernel


def _make_aggregate_kernel(num_nodes, dout, chunks_per_tile):
    half = num_nodes // NC
    acc_rows = -(-(half + NS) // (NS * 8)) * (NS * 8)
    zrows = acc_rows // NS

    mesh = plsc.VectorSubcoreMesh(
        core_axis_name="c", subcore_axis_name="s",
        num_cores=NC, num_subcores=NS)

    G = 16
    ngroups = chunks_per_tile // G
    ge = G * LANES
    CH = LANES
    cap = ge + CH
    crows = -(-cap // CH)

    last_rows = half - (NS - 1) * zrows
    assert 0 < last_rows <= zrows and last_rows % 8 == 0

    @functools.partial(
        pl.kernel,
        mesh=mesh,
        out_type=jax.ShapeDtypeStruct((num_nodes, dout), jnp.float32),
        compiler_params=pltpu.CompilerParams(needs_layout_passes=False),
        scratch_types=[
            pltpu.VMEM((2, G, LANES), jnp.int32),
            pltpu.VMEM((2, G, LANES), jnp.int32),
            pltpu.VMEM((cap,), jnp.int32),
            pltpu.VMEM((cap,), jnp.int32),
            pltpu.VMEM((crows, CH), jnp.int32),
            [pltpu.VMEM((CH, dout), jnp.float32)] * 2,
            [pltpu.SemaphoreType.DMA] * 2,
            pltpu.SemaphoreType.DMA,
            pltpu.VMEM_SHARED((acc_rows, dout), jnp.float32),
        ],
    )
    def agg_kernel(hn_hbm, src_hbm, tgt_hbm, out_hbm,
                   src_v, tgt_v, csrc, cstmp, cscat, bufs,
                   sems_g, sem_i, acc_sh):
        c = lax.axis_index("c")
        s = lax.axis_index("s")
        base = c * half
        dummy_base = half + s

        def zfill(i, _):
            def zrow(k, _):
                bufs[0][i, pl.ds(k * 16, 16)] = jnp.zeros((16,), jnp.float32)
                return 0
            lax.fori_loop(0, dout // 16, zrow, 0)
            return 0
        lax.fori_loop(0, CH, zfill, 0)
        for r in range(0, zrows, CH):
            sz = min(CH, zrows - r)
            pltpu.sync_copy(bufs[0].at[pl.ds(0, sz)],
                            acc_sh.at[pl.ds(s * zrows + r, sz)])

        def idx_start(g, slot):
            off = pl.multiple_of(g * G, 8)
            pltpu.make_async_copy(
                src_hbm.at[s, pl.ds(off, G)], src_v.at[slot], sem_i).start()
            pltpu.make_async_copy(
                tgt_hbm.at[s, pl.ds(off, G)], tgt_v.at[slot], sem_i).start()

        def idx_wait():
            pltpu.make_async_copy(
                src_hbm.at[s, pl.ds(0, G)], src_v.at[0], sem_i).wait()
            pltpu.make_async_copy(
                tgt_hbm.at[s, pl.ds(0, G)], tgt_v.at[0], sem_i).wait()

        def gather(j, q):
            pltpu.make_async_copy(
                hn_hbm.at[csrc.at[pl.ds(j * CH, CH)]], bufs[q], sems_g[q]
            ).start()

        def gwait(q):
            pltpu.make_async_copy(
                hn_hbm.at[csrc.at[pl.ds(0, CH)]], bufs[q], sems_g[q]).wait()

        def scat_add(j, q):
            pltpu.sync_copy(bufs[q], acc_sh.at[cscat.at[j]], add=True)

        idx_start(0, 0)
        idx_wait()
        plsc.subcore_barrier()

        true16 = jnp.ones((16,), jnp.bool_)
        zero16 = jnp.zeros((16,), jnp.int32)

        def group(g, _):
            slot = g & 1

            @pl.when(g + 1 < ngroups)
            def _():
                idx_start(g + 1, 1 - slot)

            def comp(i, p_v):
                for u in range(2):
                    v = i * 2 + u
                    j = v >> 3
                    off = (v & 7) * 16
                    t = tgt_v[slot, j, pl.ds(off, 16)]
                    sv = src_v[slot, j, pl.ds(off, 16)]
                    m = (t >= base) & (t < base + half)
                    pos = p_v + plsc.cumsum(m.astype(jnp.int32)) - 1
                    plsc.store_scatter(cstmp, [pos], t - base, mask=m)
                    plsc.store_scatter(csrc, [pos], sv, mask=m)
                    p_v = p_v + plsc.all_reduce_population_count(m)
                return p_v
            p_v = lax.fori_loop(0, ge // 32, comp, jnp.zeros((16,), jnp.int32))
            p = jnp.sum(p_v) >> 4

            padv = zero16 + dummy_base
            iota16 = jax.lax.iota(jnp.int32, 16)
            for q in range(CH // 16):
                padpos = p + q * 16 + iota16
                plsc.store_scatter(cstmp, [padpos], padv, mask=true16)
                plsc.store_scatter(csrc, [padpos], zero16, mask=true16)
            nch = (p + CH - 1) // CH

            def ccopy(jr, _):
                for k in range(CH // 16):
                    cscat[jr, pl.ds(k * 16, 16)] = (
                        cstmp[pl.ds(jr * CH + k * 16, 16)])
                return 0
            lax.fori_loop(0, nch, ccopy, 0)

            @pl.when(nch > 0)
            def _():
                gather(0, 0)

                def inner(i, _):
                    even = (i & 1) == 0

                    @pl.when(even)
                    def _():
                        @pl.when(i + 1 < nch)
                        def _():
                            gather(i + 1, 1)
                        gwait(0)
                        scat_add(i, 0)

                    @pl.when(jnp.logical_not(even))
                    def _():
                        @pl.when(i + 1 < nch)
                        def _():
                            gather(i + 1, 0)
                        gwait(1)
                        scat_add(i, 1)
                    return 0
                lax.fori_loop(0, nch, inner, 0)

            @pl.when(g + 1 < ngroups)
            def _():
                idx_wait()
            return 0
        lax.fori_loop(0, ngroups, group, 0)

        plsc.subcore_barrier()

        @pl.when(s < NS - 1)
        def _():
            pltpu.sync_copy(
                acc_sh.at[pl.ds(s * zrows, zrows)],
                out_hbm.at[pl.ds(c * half + s * zrows, zrows)])

        @pl.when(s == NS - 1)
        def _():
            off = (NS - 1) * zrows
            pltpu.sync_copy(
                acc_sh.at[pl.ds(off, last_rows)],
                out_hbm.at[pl.ds(c * half + off, last_rows)])

    return agg_kernel


def _linear_body(x_ref, wt_ref, b_ref, deg_ref, h_ref, hn_ref):
    x = x_ref[...]
    h = jnp.dot(x, wt_ref[...], preferred_element_type=jnp.float32) + b_ref[...]
    dis = lax.rsqrt(deg_ref[...])
    h_ref[...] = h
    hn_ref[...] = h * dis


def _tc_linear(x, wt, b2, deg_col, rb):
    n = x.shape[0]
    din = x.shape[1]
    dout = wt.shape[1]
    grid = n // rb
    return pl.pallas_call(
        _linear_body,
        grid=(grid,),
        in_specs=[
            pl.BlockSpec((rb, din), lambda i: (i, 0)),
            pl.BlockSpec((din, dout), lambda i: (0, 0)),
            pl.BlockSpec((1, dout), lambda i: (0, 0)),
            pl.BlockSpec((rb, 1), lambda i: (i, 0)),
        ],
        out_specs=[
            pl.BlockSpec((rb, dout), lambda i: (i, 0)),
            pl.BlockSpec((rb, dout), lambda i: (i, 0)),
        ],
        out_shape=[
            jax.ShapeDtypeStruct((n, dout), jnp.float32),
            jax.ShapeDtypeStruct((n, dout), jnp.float32),
        ],
    )(x, wt, b2, deg_col)


def _finish_body(h_ref, agg_ref, deg_ref, g_ref, be_ref, o_ref):
    h = h_ref[...]
    dis = lax.rsqrt(deg_ref[...])
    y = h + dis * (agg_ref[...] + h)
    mean = jnp.mean(y, axis=1, keepdims=True)
    yc = y - mean
    var = jnp.mean(yc * yc, axis=1, keepdims=True)
    o = yc * lax.rsqrt(var + 1e-5) * g_ref[...] + be_ref[...]
    o_ref[...] = jnp.maximum(o, 0.0)


def _tc_finish(h, agg, deg_col, g2, be2, rb):
    n, dout = h.shape
    grid = n // rb
    return pl.pallas_call(
        _finish_body,
        grid=(grid,),
        in_specs=[
            pl.BlockSpec((rb, dout), lambda i: (i, 0)),
            pl.BlockSpec((rb, dout), lambda i: (i, 0)),
            pl.BlockSpec((rb, 1), lambda i: (i, 0)),
            pl.BlockSpec((1, dout), lambda i: (0, 0)),
            pl.BlockSpec((1, dout), lambda i: (0, 0)),
        ],
        out_specs=pl.BlockSpec((rb, dout), lambda i: (i, 0)),
        out_shape=jax.ShapeDtypeStruct((n, dout), jnp.float32),
    )(h, agg, deg_col, g2, be2)


def kernel(node_features, edge_index, W, b, gamma, beta):
    bs, n, din = node_features.shape
    dout = W.shape[0]
    nn = bs * n
    e = edge_index.shape[2]
    be = bs * e

    ei = edge_index.astype(jnp.int32)
    offs = (jnp.arange(bs, dtype=jnp.int32) * n)[:, None]
    src = (ei[:, 0, :] + offs).reshape(-1)
    tgt = (ei[:, 1, :] + offs).reshape(-1)
    x = node_features.reshape(nn, din).astype(jnp.float32)

    cpw = -(-be // (NC * NS * LANES))
    cpw = -(-cpw // 8) * 8
    be_pad = NC * NS * cpw * LANES
    cpt = be_pad // (NS * LANES)
    pad = be_pad - be
    srcp = jnp.concatenate([src, jnp.zeros((pad,), jnp.int32)])
    tgtp = jnp.concatenate([tgt, jnp.full((pad,), nn, jnp.int32)])

    hist_n = ((nn + 1 + 255) // 256) * 256

    degree_kernel = _make_degree_kernel(nn, hist_n, cpw)
    hist = degree_kernel(tgtp.reshape(NC * NS, cpw, LANES))
    deg_col = (hist[0, :nn] + hist[1, :nn] + 1.0).reshape(nn, 1)

    rb = 1000
    h, hn = _tc_linear(x, W.T, b.reshape(1, dout), deg_col, rb)

    agg_kernel = _make_aggregate_kernel(nn, dout, cpt)
    agg = agg_kernel(hn,
                     srcp.reshape(NS, cpt, LANES),
                     tgtp.reshape(NS, cpt, LANES))

    out = _tc_finish(h, agg, deg_col,
                     gamma.reshape(1, dout), beta.reshape(1, dout), rb)
    return out.reshape(bs, n, dout)


# speedup vs baseline: 1.8383x; 1.0061x over previous
import functools

import jax
import jax.numpy as jnp
from jax import lax
from jax.experimental import pallas as pl
from jax.experimental.pallas import tpu as pltpu
from jax.experimental.pallas import tpu_sc as plsc

NC = 2
NS = 16
LANES = 128


def _make_degree_kernel(num_nodes, hist_n, chunks_per_worker):
    mesh = plsc.VectorSubcoreMesh(
        core_axis_name="c", subcore_axis_name="s",
        num_cores=NC, num_subcores=NS)

    zslice = hist_n // NS

    @functools.partial(
        pl.kernel,
        mesh=mesh,
        out_type=jax.ShapeDtypeStruct((NC, hist_n), jnp.float32),
        scratch_types=[
            pltpu.VMEM((chunks_per_worker, LANES), jnp.int32),
            pltpu.VMEM((LANES,), jnp.float32),
            pltpu.VMEM((zslice,), jnp.float32),
            pltpu.VMEM_SHARED((hist_n,), jnp.float32),
        ],
    )
    def degree_kernel(tgt_hbm, out_hbm, tgt_v, ones_v, zer_v, hist_sh):
        c = lax.axis_index("c")
        s = lax.axis_index("s")
        wid = c * NS + s

        def fill(i, _):
            ones_v[pl.ds(i * 16, 16)] = jnp.ones((16,), jnp.float32)
            return 0
        lax.fori_loop(0, LANES // 16, fill, 0)

        def zfill(i, _):
            zer_v[pl.ds(i * 16, 16)] = jnp.zeros((16,), jnp.float32)
            return 0
        lax.fori_loop(0, zslice // 16, zfill, 0)
        pltpu.sync_copy(zer_v, hist_sh.at[pl.ds(s * zslice, zslice)])

        pltpu.sync_copy(tgt_hbm.at[wid], tgt_v)
        plsc.subcore_barrier()

        def body(j, _):
            pltpu.sync_copy(ones_v, hist_sh.at[tgt_v.at[j]], add=True)
            return 0
        lax.fori_loop(0, chunks_per_worker, body, 0)

        plsc.subcore_barrier()

        @pl.when(s == 0)
        def _():
            pltpu.sync_copy(hist_sh, out_hbm.at[c])

    return degree_kernel


def _make_aggregate_kernel(num_nodes, dout, chunks_per_tile):
    half = num_nodes // NC
    acc_rows = -(-(half + NS) // (NS * 8)) * (NS * 8)
    zrows = acc_rows // NS

    mesh = plsc.VectorSubcoreMesh(
        core_axis_name="c", subcore_axis_name="s",
        num_cores=NC, num_subcores=NS)

    G = 16
    ngroups = chunks_per_tile // G
    ge = G * LANES
    CH = LANES
    cap = ge + CH
    crows = -(-cap // CH)

    last_rows = half - (NS - 1) * zrows
    assert 0 < last_rows <= zrows and last_rows % 8 == 0

    @functools.partial(
        pl.kernel,
        mesh=mesh,
        out_type=jax.ShapeDtypeStruct((num_nodes, dout), jnp.float32),
        compiler_params=pltpu.CompilerParams(needs_layout_passes=False),
        scratch_types=[
            pltpu.VMEM((2, G, LANES), jnp.int32),
            pltpu.VMEM((2, G, LANES), jnp.int32),
            pltpu.VMEM((cap,), jnp.int32),
            pltpu.VMEM((cap,), jnp.int32),
            pltpu.VMEM((crows, CH), jnp.int32),
            [pltpu.VMEM((CH, dout), jnp.float32)] * 2,
            [pltpu.SemaphoreType.DMA] * 2,
            pltpu.SemaphoreType.DMA,
            pltpu.VMEM_SHARED((acc_rows, dout), jnp.float32),
        ],
    )
    def agg_kernel(hn_hbm, src_hbm, tgt_hbm, out_hbm,
                   src_v, tgt_v, csrc, cstmp, cscat, bufs,
                   sems_g, sem_i, acc_sh):
        c = lax.axis_index("c")
        s = lax.axis_index("s")
        base = c * half
        dummy_base = half + s

        def zfill(i, _):
            def zrow(k, _):
                bufs[0][i, pl.ds(k * 16, 16)] = jnp.zeros((16,), jnp.float32)
                return 0
            lax.fori_loop(0, dout // 16, zrow, 0)
            return 0
        lax.fori_loop(0, CH, zfill, 0)
        for r in range(0, zrows, CH):
            sz = min(CH, zrows - r)
            pltpu.sync_copy(bufs[0].at[pl.ds(0, sz)],
                            acc_sh.at[pl.ds(s * zrows + r, sz)])

        def idx_start(g, slot):
            off = pl.multiple_of(g * G, 8)
            pltpu.make_async_copy(
                src_hbm.at[s, pl.ds(off, G)], src_v.at[slot], sem_i).start()
            pltpu.make_async_copy(
                tgt_hbm.at[s, pl.ds(off, G)], tgt_v.at[slot], sem_i).start()

        def idx_wait():
            pltpu.make_async_copy(
                src_hbm.at[s, pl.ds(0, G)], src_v.at[0], sem_i).wait()
            pltpu.make_async_copy(
                tgt_hbm.at[s, pl.ds(0, G)], tgt_v.at[0], sem_i).wait()

        def gather(j, q):
            pltpu.make_async_copy(
                hn_hbm.at[csrc.at[pl.ds(j * CH, CH)]], bufs[q], sems_g[q]
            ).start()

        def gwait(q):
            pltpu.make_async_copy(
                hn_hbm.at[csrc.at[pl.ds(0, CH)]], bufs[q], sems_g[q]).wait()

        def scat_add(j, q):
            pltpu.sync_copy(bufs[q], acc_sh.at[cscat.at[j]], add=True)

        idx_start(0, 0)
        idx_wait()
        plsc.subcore_barrier()

        true16 = jnp.ones((16,), jnp.bool_)
        zero16 = jnp.zeros((16,), jnp.int32)

        def group(g, _):
            slot = g & 1

            @pl.when(g + 1 < ngroups)
            def _():
                idx_start(g + 1, 1 - slot)

            def comp(i, p_v):
                for u in range(2):
                    v = i * 2 + u
                    j = v >> 3
                    off = (v & 7) * 16
                    t = tgt_v[slot, j, pl.ds(off, 16)]
                    sv = src_v[slot, j, pl.ds(off, 16)]
                    m = (t >= base) & (t < base + half)
                    pos = p_v + plsc.cumsum(m.astype(jnp.int32)) - 1
                    plsc.store_scatter(cstmp, [pos], t - base, mask=m)
                    plsc.store_scatter(csrc, [pos], sv, mask=m)
                    p_v = p_v + plsc.all_reduce_population_count(m)
                return p_v
            p_v = lax.fori_loop(0, ge // 32, comp, jnp.zeros((16,), jnp.int32))
            p = jnp.sum(p_v) >> 4

            padv = zero16 + dummy_base
            iota16 = jax.lax.iota(jnp.int32, 16)
            for q in range(CH // 16):
                padpos = p + q * 16 + iota16
                plsc.store_scatter(cstmp, [padpos], padv, mask=true16)
                plsc.store_scatter(csrc, [padpos], zero16, mask=true16)
            nch = (p + CH - 1) // CH

            def ccopy(jr, _):
                for k in range(CH // 16):
                    cscat[jr, pl.ds(k * 16, 16)] = (
                        cstmp[pl.ds(jr * CH + k * 16, 16)])
                return 0
            lax.fori_loop(0, nch, ccopy, 0)

            @pl.when(nch > 0)
            def _():
                gather(0, 0)

                def inner(i, _):
                    even = (i & 1) == 0

                    @pl.when(even)
                    def _():
                        @pl.when(i + 1 < nch)
                        def _():
                            gather(i + 1, 1)
                        gwait(0)
                        scat_add(i, 0)

                    @pl.when(jnp.logical_not(even))
                    def _():
                        @pl.when(i + 1 < nch)
                        def _():
                            gather(i + 1, 0)
                        gwait(1)
                        scat_add(i, 1)
                    return 0
                lax.fori_loop(0, nch, inner, 0)

            @pl.when(g + 1 < ngroups)
            def _():
                idx_wait()
            return 0
        lax.fori_loop(0, ngroups, group, 0)

        plsc.subcore_barrier()

        @pl.when(s < NS - 1)
        def _():
            pltpu.sync_copy(
                acc_sh.at[pl.ds(s * zrows, zrows)],
                out_hbm.at[pl.ds(c * half + s * zrows, zrows)])

        @pl.when(s == NS - 1)
        def _():
            off = (NS - 1) * zrows
            pltpu.sync_copy(
                acc_sh.at[pl.ds(off, last_rows)],
                out_hbm.at[pl.ds(c * half + off, last_rows)])

    return agg_kernel


def _linear_body(x_ref, wt_ref, b_ref, deg_ref, hn_ref):
    x = x_ref[...]
    h = jnp.dot(x, wt_ref[...], preferred_element_type=jnp.float32) + b_ref[...]
    hn_ref[...] = h * lax.rsqrt(deg_ref[...])


def _tc_linear(x, wt, b2, deg_col, rb):
    n = x.shape[0]
    din = x.shape[1]
    dout = wt.shape[1]
    grid = n // rb
    return pl.pallas_call(
        _linear_body,
        grid=(grid,),
        in_specs=[
            pl.BlockSpec((rb, din), lambda i: (i, 0)),
            pl.BlockSpec((din, dout), lambda i: (0, 0)),
            pl.BlockSpec((1, dout), lambda i: (0, 0)),
            pl.BlockSpec((rb, 1), lambda i: (i, 0)),
        ],
        out_specs=pl.BlockSpec((rb, dout), lambda i: (i, 0)),
        out_shape=jax.ShapeDtypeStruct((n, dout), jnp.float32),
    )(x, wt, b2, deg_col)


def _finish_body(hn_ref, agg_ref, deg_ref, g_ref, be_ref, o_ref):
    deg = deg_ref[...]
    dis = lax.rsqrt(deg)
    h = hn_ref[...] * jnp.sqrt(deg)
    y = h + dis * (agg_ref[...] + h)
    mean = jnp.mean(y, axis=1, keepdims=True)
    yc = y - mean
    var = jnp.mean(yc * yc, axis=1, keepdims=True)
    o = yc * lax.rsqrt(var + 1e-5) * g_ref[...] + be_ref[...]
    o_ref[...] = jnp.maximum(o, 0.0)


def _tc_finish(h, agg, deg_col, g2, be2, rb):
    n, dout = h.shape
    grid = n // rb
    return pl.pallas_call(
        _finish_body,
        grid=(grid,),
        in_specs=[
            pl.BlockSpec((rb, dout), lambda i: (i, 0)),
            pl.BlockSpec((rb, dout), lambda i: (i, 0)),
            pl.BlockSpec((rb, 1), lambda i: (i, 0)),
            pl.BlockSpec((1, dout), lambda i: (0, 0)),
            pl.BlockSpec((1, dout), lambda i: (0, 0)),
        ],
        out_specs=pl.BlockSpec((rb, dout), lambda i: (i, 0)),
        out_shape=jax.ShapeDtypeStruct((n, dout), jnp.float32),
    )(h, agg, deg_col, g2, be2)


def kernel(node_features, edge_index, W, b, gamma, beta):
    bs, n, din = node_features.shape
    dout = W.shape[0]
    nn = bs * n
    e = edge_index.shape[2]
    be = bs * e

    ei = edge_index.astype(jnp.int32)
    offs = (jnp.arange(bs, dtype=jnp.int32) * n)[:, None]
    src = (ei[:, 0, :] + offs).reshape(-1)
    tgt = (ei[:, 1, :] + offs).reshape(-1)
    x = node_features.reshape(nn, din).astype(jnp.float32)

    cpw = -(-be // (NC * NS * LANES))
    cpw = -(-cpw // 8) * 8
    be_pad = NC * NS * cpw * LANES
    cpt = be_pad // (NS * LANES)
    pad = be_pad - be
    srcp = jnp.concatenate([src, jnp.zeros((pad,), jnp.int32)])
    tgtp = jnp.concatenate([tgt, jnp.full((pad,), nn, jnp.int32)])

    hist_n = ((nn + 1 + 255) // 256) * 256

    degree_kernel = _make_degree_kernel(nn, hist_n, cpw)
    hist = degree_kernel(tgtp.reshape(NC * NS, cpw, LANES))
    deg_col = (hist[0, :nn] + hist[1, :nn] + 1.0).reshape(nn, 1)

    rb = 1000
    hn = _tc_linear(x, W.T, b.reshape(1, dout), deg_col, rb)

    agg_kernel = _make_aggregate_kernel(nn, dout, cpt)
    agg = agg_kernel(hn,
                     srcp.reshape(NS, cpt, LANES),
                     tgtp.reshape(NS, cpt, LANES))

    out = _tc_finish(hn, agg, deg_col,
                     gamma.reshape(1, dout), beta.reshape(1, dout), rb)
    return out.reshape(bs, n, dout)


# speedup vs baseline: 1.8700x; 1.0172x over previous
import functools

import jax
import jax.numpy as jnp
from jax import lax
from jax.experimental import pallas as pl
from jax.experimental.pallas import tpu as pltpu
from jax.experimental.pallas import tpu_sc as plsc

NC = 2
NS = 16
LANES = 128


def _make_degree_kernel(num_nodes, hist_n, chunks_per_worker):
    mesh = plsc.VectorSubcoreMesh(
        core_axis_name="c", subcore_axis_name="s",
        num_cores=NC, num_subcores=NS)

    zslice = hist_n // NS

    @functools.partial(
        pl.kernel,
        mesh=mesh,
        out_type=jax.ShapeDtypeStruct((NC, hist_n), jnp.float32),
        scratch_types=[
            pltpu.VMEM((chunks_per_worker, LANES), jnp.int32),
            pltpu.VMEM((LANES,), jnp.float32),
            pltpu.VMEM((zslice,), jnp.float32),
            pltpu.VMEM_SHARED((hist_n,), jnp.float32),
        ],
    )
    def degree_kernel(tgt_hbm, out_hbm, tgt_v, ones_v, zer_v, hist_sh):
        c = lax.axis_index("c")
        s = lax.axis_index("s")
        wid = c * NS + s

        def fill(i, _):
            ones_v[pl.ds(i * 16, 16)] = jnp.ones((16,), jnp.float32)
            return 0
        lax.fori_loop(0, LANES // 16, fill, 0)

        def zfill(i, _):
            zer_v[pl.ds(i * 16, 16)] = jnp.zeros((16,), jnp.float32)
            return 0
        lax.fori_loop(0, zslice // 16, zfill, 0)
        pltpu.sync_copy(zer_v, hist_sh.at[pl.ds(s * zslice, zslice)])

        pltpu.sync_copy(tgt_hbm.at[wid], tgt_v)
        plsc.subcore_barrier()

        def body(j, _):
            pltpu.sync_copy(ones_v, hist_sh.at[tgt_v.at[j]], add=True)
            return 0
        lax.fori_loop(0, chunks_per_worker, body, 0)

        plsc.subcore_barrier()

        @pl.when(s == 0)
        def _():
            pltpu.sync_copy(hist_sh, out_hbm.at[c])

    return degree_kernel


def _make_aggregate_kernel(num_nodes, dout, chunks_per_tile):
    half = num_nodes // NC
    acc_rows = -(-(half + NS) // (NS * 8)) * (NS * 8)
    zrows = acc_rows // NS

    mesh = plsc.VectorSubcoreMesh(
        core_axis_name="c", subcore_axis_name="s",
        num_cores=NC, num_subcores=NS)

    G = 16
    ngroups = chunks_per_tile // G
    ge = G * LANES
    CH = LANES
    cap = ge + CH
    crows = -(-cap // CH)

    last_rows = half - (NS - 1) * zrows
    assert 0 < last_rows <= zrows and last_rows % 8 == 0

    @functools.partial(
        pl.kernel,
        mesh=mesh,
        out_type=jax.ShapeDtypeStruct((num_nodes, dout), jnp.float32),
        compiler_params=pltpu.CompilerParams(needs_layout_passes=False),
        scratch_types=[
            pltpu.VMEM((2, G, LANES), jnp.int32),
            pltpu.VMEM((2, G, LANES), jnp.int32),
            pltpu.VMEM((cap,), jnp.int32),
            pltpu.VMEM((cap,), jnp.int32),
            pltpu.VMEM((crows, CH), jnp.int32),
            [pltpu.VMEM((CH, dout), jnp.float32)] * 2,
            [pltpu.SemaphoreType.DMA] * 2,
            pltpu.SemaphoreType.DMA,
            pltpu.VMEM_SHARED((acc_rows, dout), jnp.float32),
        ],
    )
    def agg_kernel(hn_hbm, src_hbm, tgt_hbm, out_hbm,
                   src_v, tgt_v, csrc, cstmp, cscat, bufs,
                   sems_g, sem_i, acc_sh):
        c = lax.axis_index("c")
        s = lax.axis_index("s")
        base = c * half
        dummy_base = half + s

        def zfill(i, _):
            def zrow(k, _):
                bufs[0][i, pl.ds(k * 16, 16)] = jnp.zeros((16,), jnp.float32)
                return 0
            lax.fori_loop(0, dout // 16, zrow, 0)
            return 0
        lax.fori_loop(0, CH, zfill, 0)
        for r in range(0, zrows, CH):
            sz = min(CH, zrows - r)
            pltpu.sync_copy(bufs[0].at[pl.ds(0, sz)],
                            acc_sh.at[pl.ds(s * zrows + r, sz)])

        def idx_start(g, slot):
            off = pl.multiple_of(g * G, 8)
            pltpu.make_async_copy(
                src_hbm.at[s, pl.ds(off, G)], src_v.at[slot], sem_i).start()
            pltpu.make_async_copy(
                tgt_hbm.at[s, pl.ds(off, G)], tgt_v.at[slot], sem_i).start()

        def idx_wait():
            pltpu.make_async_copy(
                src_hbm.at[s, pl.ds(0, G)], src_v.at[0], sem_i).wait()
            pltpu.make_async_copy(
                tgt_hbm.at[s, pl.ds(0, G)], tgt_v.at[0], sem_i).wait()

        def gather(j, q):
            pltpu.make_async_copy(
                hn_hbm.at[csrc.at[pl.ds(j * CH, CH)]], bufs[q], sems_g[q]
            ).start()

        def gwait(q):
            pltpu.make_async_copy(
                hn_hbm.at[csrc.at[pl.ds(0, CH)]], bufs[q], sems_g[q]).wait()

        def scat_add(j, q):
            pltpu.sync_copy(bufs[q], acc_sh.at[cscat.at[j]], add=True)

        idx_start(0, 0)
        idx_wait()
        plsc.subcore_barrier()

        true16 = jnp.ones((16,), jnp.bool_)
        zero16 = jnp.zeros((16,), jnp.int32)

        def group(g, _):
            slot = g & 1

            @pl.when(g + 1 < ngroups)
            def _():
                idx_start(g + 1, 1 - slot)

            def comp(i, p_v):
                for u in range(2):
                    v = i * 2 + u
                    j = v >> 3
                    off = (v & 7) * 16
                    t = tgt_v[slot, j, pl.ds(off, 16)]
                    sv = src_v[slot, j, pl.ds(off, 16)]
                    m = (t >= base) & (t < base + half)
                    pos = p_v + plsc.cumsum(m.astype(jnp.int32)) - 1
                    plsc.store_scatter(cstmp, [pos], t - base, mask=m)
                    plsc.store_scatter(csrc, [pos], sv, mask=m)
                    p_v = p_v + plsc.all_reduce_population_count(m)
                return p_v
            p_v = lax.fori_loop(0, ge // 32, comp, jnp.zeros((16,), jnp.int32))
            p = jnp.sum(p_v) >> 4

            padv = zero16 + dummy_base
            iota16 = jax.lax.iota(jnp.int32, 16)
            for q in range(CH // 16):
                padpos = p + q * 16 + iota16
                plsc.store_scatter(cstmp, [padpos], padv, mask=true16)
                plsc.store_scatter(csrc, [padpos], zero16, mask=true16)
            nch = (p + CH - 1) // CH

            @pl.when(nch > 0)
            def _():
                gather(0, 0)

            @pl.when(nch > 1)
            def _():
                gather(1, 1)

            def ccopy(jr, _):
                for k in range(CH // 16):
                    cscat[jr, pl.ds(k * 16, 16)] = (
                        cstmp[pl.ds(jr * CH + k * 16, 16)])
                return 0
            lax.fori_loop(0, nch, ccopy, 0)

            def inner(i, _):
                even = (i & 1) == 0

                @pl.when(even)
                def _():
                    gwait(0)
                    scat_add(i, 0)

                    @pl.when(i + 2 < nch)
                    def _():
                        gather(i + 2, 0)

                @pl.when(jnp.logical_not(even))
                def _():
                    gwait(1)
                    scat_add(i, 1)

                    @pl.when(i + 2 < nch)
                    def _():
                        gather(i + 2, 1)
                return 0
            lax.fori_loop(0, nch, inner, 0)

            @pl.when(g + 1 < ngroups)
            def _():
                idx_wait()
            return 0
        lax.fori_loop(0, ngroups, group, 0)

        plsc.subcore_barrier()

        @pl.when(s < NS - 1)
        def _():
            pltpu.sync_copy(
                acc_sh.at[pl.ds(s * zrows, zrows)],
                out_hbm.at[pl.ds(c * half + s * zrows, zrows)])

        @pl.when(s == NS - 1)
        def _():
            off = (NS - 1) * zrows
            pltpu.sync_copy(
                acc_sh.at[pl.ds(off, last_rows)],
                out_hbm.at[pl.ds(c * half + off, last_rows)])

    return agg_kernel


def _linear_body(x_ref, wt_ref, b_ref, deg_ref, hn_ref):
    x = x_ref[...]
    h = jnp.dot(x, wt_ref[...], preferred_element_type=jnp.float32) + b_ref[...]
    hn_ref[...] = h * lax.rsqrt(deg_ref[...])


def _tc_linear(x, wt, b2, deg_col, rb):
    n = x.shape[0]
    din = x.shape[1]
    dout = wt.shape[1]
    grid = n // rb
    return pl.pallas_call(
        _linear_body,
        grid=(grid,),
        in_specs=[
            pl.BlockSpec((rb, din), lambda i: (i, 0)),
            pl.BlockSpec((din, dout), lambda i: (0, 0)),
            pl.BlockSpec((1, dout), lambda i: (0, 0)),
            pl.BlockSpec((rb, 1), lambda i: (i, 0)),
        ],
        out_specs=pl.BlockSpec((rb, dout), lambda i: (i, 0)),
        out_shape=jax.ShapeDtypeStruct((n, dout), jnp.float32),
    )(x, wt, b2, deg_col)


def _finish_body(hn_ref, agg_ref, deg_ref, g_ref, be_ref, o_ref):
    deg = deg_ref[...]
    dis = lax.rsqrt(deg)
    h = hn_ref[...] * jnp.sqrt(deg)
    y = h + dis * (agg_ref[...] + h)
    mean = jnp.mean(y, axis=1, keepdims=True)
    yc = y - mean
    var = jnp.mean(yc * yc, axis=1, keepdims=True)
    o = yc * lax.rsqrt(var + 1e-5) * g_ref[...] + be_ref[...]
    o_ref[...] = jnp.maximum(o, 0.0)


def _tc_finish(h, agg, deg_col, g2, be2, rb):
    n, dout = h.shape
    grid = n // rb
    return pl.pallas_call(
        _finish_body,
        grid=(grid,),
        in_specs=[
            pl.BlockSpec((rb, dout), lambda i: (i, 0)),
            pl.BlockSpec((rb, dout), lambda i: (i, 0)),
            pl.BlockSpec((rb, 1), lambda i: (i, 0)),
            pl.BlockSpec((1, dout), lambda i: (0, 0)),
            pl.BlockSpec((1, dout), lambda i: (0, 0)),
        ],
        out_specs=pl.BlockSpec((rb, dout), lambda i: (i, 0)),
        out_shape=jax.ShapeDtypeStruct((n, dout), jnp.float32),
    )(h, agg, deg_col, g2, be2)


def kernel(node_features, edge_index, W, b, gamma, beta):
    bs, n, din = node_features.shape
    dout = W.shape[0]
    nn = bs * n
    e = edge_index.shape[2]
    be = bs * e

    ei = edge_index.astype(jnp.int32)
    offs = (jnp.arange(bs, dtype=jnp.int32) * n)[:, None]
    src = (ei[:, 0, :] + offs).reshape(-1)
    tgt = (ei[:, 1, :] + offs).reshape(-1)
    x = node_features.reshape(nn, din).astype(jnp.float32)

    cpw = -(-be // (NC * NS * LANES))
    cpw = -(-cpw // 8) * 8
    be_pad = NC * NS * cpw * LANES
    cpt = be_pad // (NS * LANES)
    pad = be_pad - be
    srcp = jnp.concatenate([src, jnp.zeros((pad,), jnp.int32)])
    tgtp = jnp.concatenate([tgt, jnp.full((pad,), nn, jnp.int32)])

    hist_n = ((nn + 1 + 255) // 256) * 256

    degree_kernel = _make_degree_kernel(nn, hist_n, cpw)
    hist = degree_kernel(tgtp.reshape(NC * NS, cpw, LANES))
    deg_col = (hist[0, :nn] + hist[1, :nn] + 1.0).reshape(nn, 1)

    rb = 1000
    hn = _tc_linear(x, W.T, b.reshape(1, dout), deg_col, rb)

    agg_kernel = _make_aggregate_kernel(nn, dout, cpt)
    agg = agg_kernel(hn,
                     srcp.reshape(NS, cpt, LANES),
                     tgtp.reshape(NS, cpt, LANES))

    out = _tc_finish(hn, agg, deg_col,
                     gamma.reshape(1, dout), beta.reshape(1, dout), rb)
    return out.reshape(bs, n, dout)


# speedup vs baseline: 1.8707x; 1.0004x over previous
import functools

import jax
import jax.numpy as jnp
from jax import lax
from jax.experimental import pallas as pl
from jax.experimental.pallas import tpu as pltpu
from jax.experimental.pallas import tpu_sc as plsc

NC = 2
NS = 16
LANES = 128


def _make_degree_kernel(num_nodes, hist_n, chunks_per_worker):
    mesh = plsc.VectorSubcoreMesh(
        core_axis_name="c", subcore_axis_name="s",
        num_cores=NC, num_subcores=NS)

    zslice = hist_n // NS

    @functools.partial(
        pl.kernel,
        mesh=mesh,
        out_type=jax.ShapeDtypeStruct((NC, hist_n), jnp.float32),
        scratch_types=[
            pltpu.VMEM((chunks_per_worker, LANES), jnp.int32),
            pltpu.VMEM((LANES,), jnp.float32),
            pltpu.VMEM((zslice,), jnp.float32),
            pltpu.VMEM_SHARED((hist_n,), jnp.float32),
        ],
    )
    def degree_kernel(tgt_hbm, out_hbm, tgt_v, ones_v, zer_v, hist_sh):
        c = lax.axis_index("c")
        s = lax.axis_index("s")
        wid = c * NS + s

        def fill(i, _):
            ones_v[pl.ds(i * 16, 16)] = jnp.ones((16,), jnp.float32)
            return 0
        lax.fori_loop(0, LANES // 16, fill, 0)

        def zfill(i, _):
            zer_v[pl.ds(i * 16, 16)] = jnp.zeros((16,), jnp.float32)
            return 0
        lax.fori_loop(0, zslice // 16, zfill, 0)
        pltpu.sync_copy(zer_v, hist_sh.at[pl.ds(s * zslice, zslice)])

        pltpu.sync_copy(tgt_hbm.at[wid], tgt_v)
        plsc.subcore_barrier()

        def body(j, _):
            pltpu.sync_copy(ones_v, hist_sh.at[tgt_v.at[j]], add=True)
            return 0
        lax.fori_loop(0, chunks_per_worker, body, 0)

        plsc.subcore_barrier()

        @pl.when(s == 0)
        def _():
            pltpu.sync_copy(hist_sh, out_hbm.at[c])

    return degree_kernel


def _make_aggregate_kernel(num_nodes, dout, chunks_per_tile):
    half = num_nodes // NC
    acc_rows = -(-(half + NS) // (NS * 8)) * (NS * 8)
    zrows = acc_rows // NS

    mesh = plsc.VectorSubcoreMesh(
        core_axis_name="c", subcore_axis_name="s",
        num_cores=NC, num_subcores=NS)

    G = 16
    ngroups = chunks_per_tile // G
    ge = G * LANES
    CH = LANES
    cap = ge + CH
    crows = -(-cap // CH)

    last_rows = half - (NS - 1) * zrows
    assert 0 < last_rows <= zrows and last_rows % 8 == 0

    @functools.partial(
        pl.kernel,
        mesh=mesh,
        out_type=jax.ShapeDtypeStruct((num_nodes, dout), jnp.float32),
        compiler_params=pltpu.CompilerParams(needs_layout_passes=False),
        scratch_types=[
            pltpu.VMEM((2, G, LANES), jnp.int32),
            pltpu.VMEM((2, G, LANES), jnp.int32),
            pltpu.VMEM((cap,), jnp.int32),
            pltpu.VMEM((cap,), jnp.int32),
            pltpu.VMEM((crows, CH), jnp.int32),
            [pltpu.VMEM((CH, dout), jnp.float32)] * 2,
            [pltpu.SemaphoreType.DMA] * 2,
            pltpu.SemaphoreType.DMA,
            pltpu.VMEM_SHARED((acc_rows, dout), jnp.float32),
        ],
    )
    def agg_kernel(hn_hbm, src_hbm, tgt_hbm, out_hbm,
                   src_v, tgt_v, csrc, cstmp, cscat, bufs,
                   sems_g, sem_i, acc_sh):
        c = lax.axis_index("c")
        s = lax.axis_index("s")
        base = c * half
        dummy_base = half + s

        def zfill(i, _):
            def zrow(k, _):
                bufs[0][i, pl.ds(k * 16, 16)] = jnp.zeros((16,), jnp.float32)
                return 0
            lax.fori_loop(0, dout // 16, zrow, 0)
            return 0
        lax.fori_loop(0, CH, zfill, 0)
        for r in range(0, zrows, CH):
            sz = min(CH, zrows - r)
            pltpu.sync_copy(bufs[0].at[pl.ds(0, sz)],
                            acc_sh.at[pl.ds(s * zrows + r, sz)])

        def idx_start(g, slot):
            off = pl.multiple_of(g * G, 8)
            pltpu.make_async_copy(
                src_hbm.at[s, pl.ds(off, G)], src_v.at[slot], sem_i).start()
            pltpu.make_async_copy(
                tgt_hbm.at[s, pl.ds(off, G)], tgt_v.at[slot], sem_i).start()

        def idx_wait():
            pltpu.make_async_copy(
                src_hbm.at[s, pl.ds(0, G)], src_v.at[0], sem_i).wait()
            pltpu.make_async_copy(
                tgt_hbm.at[s, pl.ds(0, G)], tgt_v.at[0], sem_i).wait()

        def gather(j, q):
            pltpu.make_async_copy(
                hn_hbm.at[csrc.at[pl.ds(j * CH, CH)]], bufs[q], sems_g[q]
            ).start()

        def gwait(q):
            pltpu.make_async_copy(
                hn_hbm.at[csrc.at[pl.ds(0, CH)]], bufs[q], sems_g[q]).wait()

        def scat_add(j, q):
            pltpu.sync_copy(bufs[q], acc_sh.at[cscat.at[j]], add=True)

        idx_start(0, 0)
        idx_wait()
        plsc.subcore_barrier()

        true16 = jnp.ones((16,), jnp.bool_)
        zero16 = jnp.zeros((16,), jnp.int32)

        def group(g, _):
            slot = g & 1

            @pl.when(g + 1 < ngroups)
            def _():
                idx_start(g + 1, 1 - slot)

            def comp(i, p_v):
                for u in range(4):
                    v = i * 4 + u
                    j = v >> 3
                    off = (v & 7) * 16
                    t = tgt_v[slot, j, pl.ds(off, 16)]
                    sv = src_v[slot, j, pl.ds(off, 16)]
                    m = (t >= base) & (t < base + half)
                    pos = p_v + plsc.cumsum(m.astype(jnp.int32)) - 1
                    plsc.store_scatter(cstmp, [pos], t - base, mask=m)
                    plsc.store_scatter(csrc, [pos], sv, mask=m)
                    p_v = p_v + plsc.all_reduce_population_count(m)
                return p_v
            p_v = lax.fori_loop(0, ge // 64, comp, jnp.zeros((16,), jnp.int32))
            p = jnp.sum(p_v) >> 4

            padv = zero16 + dummy_base
            iota16 = jax.lax.iota(jnp.int32, 16)
            for q in range(CH // 16):
                padpos = p + q * 16 + iota16
                plsc.store_scatter(cstmp, [padpos], padv, mask=true16)
                plsc.store_scatter(csrc, [padpos], zero16, mask=true16)
            nch = (p + CH - 1) // CH

            @pl.when(nch > 0)
            def _():
                gather(0, 0)

            @pl.when(nch > 1)
            def _():
                gather(1, 1)

            def ccopy(jr, _):
                for k in range(CH // 16):
                    cscat[jr, pl.ds(k * 16, 16)] = (
                        cstmp[pl.ds(jr * CH + k * 16, 16)])
                return 0
            lax.fori_loop(0, nch, ccopy, 0)

            def inner(i, _):
                even = (i & 1) == 0

                @pl.when(even)
                def _():
                    gwait(0)
                    scat_add(i, 0)

                    @pl.when(i + 2 < nch)
                    def _():
                        gather(i + 2, 0)

                @pl.when(jnp.logical_not(even))
                def _():
                    gwait(1)
                    scat_add(i, 1)

                    @pl.when(i + 2 < nch)
                    def _():
                        gather(i + 2, 1)
                return 0
            lax.fori_loop(0, nch, inner, 0)

            @pl.when(g + 1 < ngroups)
            def _():
                idx_wait()
            return 0
        lax.fori_loop(0, ngroups, group, 0)

        plsc.subcore_barrier()

        @pl.when(s < NS - 1)
        def _():
            pltpu.sync_copy(
                acc_sh.at[pl.ds(s * zrows, zrows)],
                out_hbm.at[pl.ds(c * half + s * zrows, zrows)])

        @pl.when(s == NS - 1)
        def _():
            off = (NS - 1) * zrows
            pltpu.sync_copy(
                acc_sh.at[pl.ds(off, last_rows)],
                out_hbm.at[pl.ds(c * half + off, last_rows)])

    return agg_kernel


def _linear_body(x_ref, wt_ref, b_ref, deg_ref, hn_ref):
    x = x_ref[...]
    h = jnp.dot(x, wt_ref[...], preferred_element_type=jnp.float32) + b_ref[...]
    hn_ref[...] = h * lax.rsqrt(deg_ref[...])


def _tc_linear(x, wt, b2, deg_col, rb):
    n = x.shape[0]
    din = x.shape[1]
    dout = wt.shape[1]
    grid = n // rb
    return pl.pallas_call(
        _linear_body,
        grid=(grid,),
        in_specs=[
            pl.BlockSpec((rb, din), lambda i: (i, 0)),
            pl.BlockSpec((din, dout), lambda i: (0, 0)),
            pl.BlockSpec((1, dout), lambda i: (0, 0)),
            pl.BlockSpec((rb, 1), lambda i: (i, 0)),
        ],
        out_specs=pl.BlockSpec((rb, dout), lambda i: (i, 0)),
        out_shape=jax.ShapeDtypeStruct((n, dout), jnp.float32),
    )(x, wt, b2, deg_col)


def _finish_body(hn_ref, agg_ref, deg_ref, g_ref, be_ref, o_ref):
    deg = deg_ref[...]
    dis = lax.rsqrt(deg)
    h = hn_ref[...] * jnp.sqrt(deg)
    y = h + dis * (agg_ref[...] + h)
    mean = jnp.mean(y, axis=1, keepdims=True)
    yc = y - mean
    var = jnp.mean(yc * yc, axis=1, keepdims=True)
    o = yc * lax.rsqrt(var + 1e-5) * g_ref[...] + be_ref[...]
    o_ref[...] = jnp.maximum(o, 0.0)


def _tc_finish(h, agg, deg_col, g2, be2, rb):
    n, dout = h.shape
    grid = n // rb
    return pl.pallas_call(
        _finish_body,
        grid=(grid,),
        in_specs=[
            pl.BlockSpec((rb, dout), lambda i: (i, 0)),
            pl.BlockSpec((rb, dout), lambda i: (i, 0)),
            pl.BlockSpec((rb, 1), lambda i: (i, 0)),
            pl.BlockSpec((1, dout), lambda i: (0, 0)),
            pl.BlockSpec((1, dout), lambda i: (0, 0)),
        ],
        out_specs=pl.BlockSpec((rb, dout), lambda i: (i, 0)),
        out_shape=jax.ShapeDtypeStruct((n, dout), jnp.float32),
    )(h, agg, deg_col, g2, be2)


def kernel(node_features, edge_index, W, b, gamma, beta):
    bs, n, din = node_features.shape
    dout = W.shape[0]
    nn = bs * n
    e = edge_index.shape[2]
    be = bs * e

    ei = edge_index.astype(jnp.int32)
    offs = (jnp.arange(bs, dtype=jnp.int32) * n)[:, None]
    src = (ei[:, 0, :] + offs).reshape(-1)
    tgt = (ei[:, 1, :] + offs).reshape(-1)
    x = node_features.reshape(nn, din).astype(jnp.float32)

    cpw = -(-be // (NC * NS * LANES))
    cpw = -(-cpw // 8) * 8
    be_pad = NC * NS * cpw * LANES
    cpt = be_pad // (NS * LANES)
    pad = be_pad - be
    srcp = jnp.concatenate([src, jnp.zeros((pad,), jnp.int32)])
    tgtp = jnp.concatenate([tgt, jnp.full((pad,), nn, jnp.int32)])

    hist_n = ((nn + 1 + 255) // 256) * 256

    degree_kernel = _make_degree_kernel(nn, hist_n, cpw)
    hist = degree_kernel(tgtp.reshape(NC * NS, cpw, LANES))
    deg_col = (hist[0, :nn] + hist[1, :nn] + 1.0).reshape(nn, 1)

    rb = 1000
    hn = _tc_linear(x, W.T, b.reshape(1, dout), deg_col, rb)

    agg_kernel = _make_aggregate_kernel(nn, dout, cpt)
    agg = agg_kernel(hn,
                     srcp.reshape(NS, cpt, LANES),
                     tgtp.reshape(NS, cpt, LANES))

    out = _tc_finish(hn, agg, deg_col,
                     gamma.reshape(1, dout), beta.reshape(1, dout), rb)
    return out.reshape(bs, n, dout)
